# Initial kernel scaffold; baseline (speedup 1.0000x reference)
import jax
import jax.numpy as jnp
from jax.experimental import pallas as pl


def kernel(scores, target):
    raise NotImplementedError("write your pallas kernel here")


# speedup vs baseline: 3.1805x; 3.1805x over previous
import functools
import math

import jax
import jax.numpy as jnp
from jax import lax
from jax.experimental import pallas as pl

EPS = 0.1
DROP_WORST_RATIO = 0.2


def _row_stats_body(scores_ref, target_ref, loss_ref, *, V, on, off, const):
    x = scores_ref[...]
    m = jnp.max(x, axis=1, keepdims=True)
    se = jnp.sum(jnp.exp(x - m), axis=1, keepdims=True)
    lse = m + jnp.log(se)
    rs = jnp.sum(x, axis=1, keepdims=True)
    t = target_ref[...]
    cols = lax.broadcasted_iota(jnp.int32, x.shape, 1)
    st = jnp.sum(jnp.where(cols == t, x, 0.0), axis=1, keepdims=True)
    loss_ref[...] = (const + lse) - off * rs - (on - off) * st


def _select_mean_body(loss_ref, out_ref, *, k):
    lv = loss_ref[...]
    u = lax.bitcast_convert_type(lv, jnp.uint32)
    key = jnp.where(u >= jnp.uint32(0x80000000), ~u, u | jnp.uint32(0x80000000))

    def body(i, prefix):
        bit = jnp.uint32(31) - i.astype(jnp.uint32)
        trial = prefix | (jnp.uint32(1) << bit)
        c = jnp.sum((key < trial).astype(jnp.int32))
        return jnp.where(c < k, trial, prefix)

    kth = lax.fori_loop(0, 32, body, jnp.uint32(0))
    below = key < kth
    cnt = jnp.sum(below.astype(jnp.int32))
    ssum = jnp.sum(jnp.where(below, lv, 0.0))
    kth_bits = jnp.where(kth >= jnp.uint32(0x80000000),
                         kth ^ jnp.uint32(0x80000000), ~kth)
    kth_val = lax.bitcast_convert_type(kth_bits, jnp.float32)
    total = ssum + (k - cnt).astype(jnp.float32) * kth_val
    out_ref[...] = jnp.broadcast_to(total / jnp.float32(k), (1, 1))


def kernel(scores, target):
    N, V = scores.shape
    on = 1.0 - EPS
    off = EPS / (V - 1)
    const = on * math.log(on) + (V - 1) * (off * math.log(off))
    k = int(N * (1.0 - DROP_WORST_RATIO))

    R = 64 if N % 64 == 0 else N
    grid = (N // R,)
    loss = pl.pallas_call(
        functools.partial(_row_stats_body, V=V, on=on, off=off, const=const),
        grid=grid,
        in_specs=[
            pl.BlockSpec((R, V), lambda i: (i, 0)),
            pl.BlockSpec((R, 1), lambda i: (i, 0)),
        ],
        out_specs=pl.BlockSpec((R, 1), lambda i: (i, 0)),
        out_shape=jax.ShapeDtypeStruct((N, 1), jnp.float32),
    )(scores, target.astype(jnp.int32).reshape(N, 1))

    loss2d = loss.reshape(N // 128, 128)
    out = pl.pallas_call(
        functools.partial(_select_mean_body, k=k),
        out_shape=jax.ShapeDtypeStruct((1, 1), jnp.float32),
    )(loss2d)
    return out.reshape(())
